# baseline (speedup 1.0000x reference)
import functools

import jax
import jax.numpy as jnp
from jax import lax
from jax.experimental import pallas as pl
from jax.experimental.pallas import tpu as pltpu
from jax.experimental.pallas import tpu_sc as plsc

B = 16
L = 2048
P = 2048
D = 1024
PBLK = 512
NBLK = P // PBLK

_INFO = plsc.get_sparse_core_info()
_NC, _NS = _INFO.num_cores, _INFO.num_subcores


@functools.partial(
    pl.kernel,
    mesh=plsc.VectorSubcoreMesh(core_axis_name="c", subcore_axis_name="s"),
    out_type=jax.ShapeDtypeStruct((B, P + L), jnp.int32),
)
def _sc_ids(pids_hbm, inp_hbm, out_hbm):
    wid = lax.axis_index("s") * _NC + lax.axis_index("c")
    b = wid // 2
    half = wid % 2

    @pl.when(half == 0)
    def _prompt():
        pltpu.sync_copy(pids_hbm, out_hbm.at[b, pl.ds(0, P)])

    @pl.when(half == 1)
    def _user():
        pltpu.sync_copy(inp_hbm.at[b], out_hbm.at[b, pl.ds(P, L)])


def _kernel_body(pids_ref, w_ref, emb_out_ref,
                 g_ref, sem0, sem1, sem2, sem3):
    sems = (sem0, sem1, sem2, sem3)
    pending = []
    for j in range(NBLK):
        base = j * PBLK
        idsblk = pids_ref[0:1, base:base + PBLK]
        rows = jax.lax.broadcasted_iota(jnp.int32, (P, PBLK), 0)
        onehot_t = (rows == idsblk).astype(jnp.float32)
        g = jax.lax.dot_general(
            onehot_t, w_ref[...], (((0,), (0,)), ((), ())),
            preferred_element_type=jnp.float32,
        )
        hit = jax.lax.dot_general(
            onehot_t, jnp.ones((P, 1), jnp.float32), (((0,), (0,)), ((), ())),
            preferred_element_type=jnp.float32,
        )
        g_ref[base:base + PBLK, :] = jnp.where(hit > 0.5, g, jnp.float32(jnp.nan))
        fired = []
        for b in range(B):
            c = pltpu.make_async_copy(
                g_ref.at[pl.ds(base, PBLK), :],
                emb_out_ref.at[b, pl.ds(base, PBLK), :],
                sems[b % 4],
            )
            c.start()
            fired.append(c)
        pending.append(fired)
        if len(pending) > 2:
            for c in pending.pop(0):
                c.wait()
    for fired in pending:
        for c in fired:
            c.wait()


@functools.partial(jax.jit)
def kernel(input_ids, prompt_ids, embed_weight):
    ids_out = _sc_ids(prompt_ids, input_ids)
    pids2 = prompt_ids.reshape(1, P)
    emb_out = pl.pallas_call(
        _kernel_body,
        in_specs=[
            pl.BlockSpec(memory_space=pltpu.MemorySpace.VMEM),
            pl.BlockSpec(memory_space=pltpu.MemorySpace.VMEM),
        ],
        out_specs=pl.BlockSpec(memory_space=pl.ANY),
        out_shape=jax.ShapeDtypeStruct((B, P, D), jnp.float32),
        scratch_shapes=[
            pltpu.VMEM((P, D), jnp.float32),
            pltpu.SemaphoreType.DMA,
            pltpu.SemaphoreType.DMA,
            pltpu.SemaphoreType.DMA,
            pltpu.SemaphoreType.DMA,
        ],
    )(pids2, embed_weight)
    return ids_out, emb_out


# speedup vs baseline: 1.2229x; 1.2229x over previous
import functools

import jax
import jax.numpy as jnp
from jax.experimental import pallas as pl
from jax.experimental.pallas import tpu as pltpu

B = 16
L = 2048
P = 2048
D = 1024
PBLK = 1024
NBLK = P // PBLK


def _kernel_body(pids_ref, inp_ref, w_ref, ids_out_ref, emb_out_ref,
                 g_ref, ids_scr_ref, sem0, sem1, sem2, sem3, ids_sem):
    sems = (sem0, sem1, sem2, sem3)
    ids_scr_ref[:, 0:P] = jnp.broadcast_to(pids_ref[...], (B, P))
    ids_scr_ref[:, P:P + L] = inp_ref[...]
    ids_copy = pltpu.make_async_copy(ids_scr_ref, ids_out_ref, ids_sem)
    ids_copy.start()

    pending = []
    for j in range(NBLK):
        base = j * PBLK
        idsblk = pids_ref[0:1, base:base + PBLK]
        rows = jax.lax.broadcasted_iota(jnp.int32, (P, PBLK), 0)
        onehot_t = (rows == idsblk).astype(jnp.float32)
        g = jax.lax.dot_general(
            onehot_t, w_ref[...], (((0,), (0,)), ((), ())),
            preferred_element_type=jnp.float32,
        )
        hit = jax.lax.dot_general(
            onehot_t, jnp.ones((P, 1), jnp.float32), (((0,), (0,)), ((), ())),
            preferred_element_type=jnp.float32,
        )
        g_ref[base:base + PBLK, :] = jnp.where(hit > 0.5, g, jnp.float32(jnp.nan))
        fired = []
        for b in range(B):
            c = pltpu.make_async_copy(
                g_ref.at[pl.ds(base, PBLK), :],
                emb_out_ref.at[b, pl.ds(base, PBLK), :],
                sems[b % 4],
            )
            c.start()
            fired.append(c)
        pending.append(fired)
        if len(pending) > 2:
            for c in pending.pop(0):
                c.wait()
    for fired in pending:
        for c in fired:
            c.wait()
    ids_copy.wait()


@functools.partial(jax.jit)
def kernel(input_ids, prompt_ids, embed_weight):
    pids2 = prompt_ids.reshape(1, P)
    ids_out, emb_out = pl.pallas_call(
        _kernel_body,
        in_specs=[
            pl.BlockSpec(memory_space=pltpu.MemorySpace.VMEM),
            pl.BlockSpec(memory_space=pltpu.MemorySpace.VMEM),
            pl.BlockSpec(memory_space=pltpu.MemorySpace.VMEM),
        ],
        out_specs=[
            pl.BlockSpec(memory_space=pl.ANY),
            pl.BlockSpec(memory_space=pl.ANY),
        ],
        out_shape=[
            jax.ShapeDtypeStruct((B, P + L), jnp.int32),
            jax.ShapeDtypeStruct((B, P, D), jnp.float32),
        ],
        scratch_shapes=[
            pltpu.VMEM((P, D), jnp.float32),
            pltpu.VMEM((B, P + L), jnp.int32),
            pltpu.SemaphoreType.DMA,
            pltpu.SemaphoreType.DMA,
            pltpu.SemaphoreType.DMA,
            pltpu.SemaphoreType.DMA,
            pltpu.SemaphoreType.DMA,
        ],
    )(pids2, input_ids, embed_weight)
    return ids_out, emb_out


# speedup vs baseline: 1.3420x; 1.0973x over previous
import functools

import jax
import jax.numpy as jnp
from jax.experimental import pallas as pl
from jax.experimental.pallas import tpu as pltpu

B = 16
L = 2048
P = 2048
D = 1024
PBLK = 256
NBLK = P // PBLK


def _kernel_body(pids_ref, inp_ref, w_ref, ids_out_ref, emb_out_ref,
                 g_ref, ids_scr_ref, sem0, sem1, sem2, sem3, ids_sem):
    sems = (sem0, sem1, sem2, sem3)
    ids_scr_ref[:, 0:P] = jnp.broadcast_to(pids_ref[...], (B, P))
    ids_scr_ref[:, P:P + L] = inp_ref[...]
    ids_copy = pltpu.make_async_copy(ids_scr_ref, ids_out_ref, ids_sem)
    ids_copy.start()

    pending = []
    for j in range(NBLK):
        base = j * PBLK
        idsblk = pids_ref[0:1, base:base + PBLK]
        rows = jax.lax.broadcasted_iota(jnp.int32, (P, PBLK), 0)
        onehot_t = (rows == idsblk).astype(jnp.float32)
        g = jax.lax.dot_general(
            onehot_t, w_ref[...], (((0,), (0,)), ((), ())),
            preferred_element_type=jnp.float32,
        )
        hit = jax.lax.dot_general(
            onehot_t, jnp.ones((P, 1), jnp.float32), (((0,), (0,)), ((), ())),
            preferred_element_type=jnp.float32,
        )
        g_ref[base:base + PBLK, :] = jnp.where(hit > 0.5, g, jnp.float32(jnp.nan))
        fired = []
        for b in range(B):
            c = pltpu.make_async_copy(
                g_ref.at[pl.ds(base, PBLK), :],
                emb_out_ref.at[b, pl.ds(base, PBLK), :],
                sems[b % 4],
            )
            c.start()
            fired.append(c)
        pending.append(fired)
        if len(pending) > 2:
            for c in pending.pop(0):
                c.wait()
    for fired in pending:
        for c in fired:
            c.wait()
    ids_copy.wait()


@functools.partial(jax.jit)
def kernel(input_ids, prompt_ids, embed_weight):
    pids2 = prompt_ids.reshape(1, P)
    ids_out, emb_out = pl.pallas_call(
        _kernel_body,
        in_specs=[
            pl.BlockSpec(memory_space=pltpu.MemorySpace.VMEM),
            pl.BlockSpec(memory_space=pltpu.MemorySpace.VMEM),
            pl.BlockSpec(memory_space=pltpu.MemorySpace.VMEM),
        ],
        out_specs=[
            pl.BlockSpec(memory_space=pl.ANY),
            pl.BlockSpec(memory_space=pl.ANY),
        ],
        out_shape=[
            jax.ShapeDtypeStruct((B, P + L), jnp.int32),
            jax.ShapeDtypeStruct((B, P, D), jnp.float32),
        ],
        scratch_shapes=[
            pltpu.VMEM((P, D), jnp.float32),
            pltpu.VMEM((B, P + L), jnp.int32),
            pltpu.SemaphoreType.DMA,
            pltpu.SemaphoreType.DMA,
            pltpu.SemaphoreType.DMA,
            pltpu.SemaphoreType.DMA,
            pltpu.SemaphoreType.DMA,
        ],
    )(pids2, input_ids, embed_weight)
    return ids_out, emb_out


# speedup vs baseline: 1.3484x; 1.0048x over previous
import functools

import jax
import jax.numpy as jnp
from jax.experimental import pallas as pl
from jax.experimental.pallas import tpu as pltpu

B = 16
L = 2048
P = 2048
D = 1024
PBLK = 128
NBLK = P // PBLK


def _kernel_body(pids_ref, inp_ref, w_ref, ids_out_ref, emb_out_ref,
                 g_ref, ids_scr_ref, sem0, sem1, sem2, sem3, ids_sem):
    sems = (sem0, sem1, sem2, sem3)
    ids_scr_ref[:, 0:P] = jnp.broadcast_to(pids_ref[...], (B, P))
    ids_scr_ref[:, P:P + L] = inp_ref[...]
    ids_copy = pltpu.make_async_copy(ids_scr_ref, ids_out_ref, ids_sem)
    ids_copy.start()

    pending = []
    for j in range(NBLK):
        base = j * PBLK
        idsblk = pids_ref[0:1, base:base + PBLK]
        rows = jax.lax.broadcasted_iota(jnp.int32, (P, PBLK), 0)
        onehot_t = (rows == idsblk).astype(jnp.float32)
        g = jax.lax.dot_general(
            onehot_t, w_ref[...], (((0,), (0,)), ((), ())),
            preferred_element_type=jnp.float32,
        )
        hit = jax.lax.dot_general(
            onehot_t, jnp.ones((P, 1), jnp.float32), (((0,), (0,)), ((), ())),
            preferred_element_type=jnp.float32,
        )
        g_ref[base:base + PBLK, :] = jnp.where(hit > 0.5, g, jnp.float32(jnp.nan))
        fired = []
        for b in range(B):
            c = pltpu.make_async_copy(
                g_ref.at[pl.ds(base, PBLK), :],
                emb_out_ref.at[b, pl.ds(base, PBLK), :],
                sems[b % 4],
            )
            c.start()
            fired.append(c)
        pending.append(fired)
        if len(pending) > 2:
            for c in pending.pop(0):
                c.wait()
    for fired in pending:
        for c in fired:
            c.wait()
    ids_copy.wait()


@functools.partial(jax.jit)
def kernel(input_ids, prompt_ids, embed_weight):
    pids2 = prompt_ids.reshape(1, P)
    ids_out, emb_out = pl.pallas_call(
        _kernel_body,
        in_specs=[
            pl.BlockSpec(memory_space=pltpu.MemorySpace.VMEM),
            pl.BlockSpec(memory_space=pltpu.MemorySpace.VMEM),
            pl.BlockSpec(memory_space=pltpu.MemorySpace.VMEM),
        ],
        out_specs=[
            pl.BlockSpec(memory_space=pl.ANY),
            pl.BlockSpec(memory_space=pl.ANY),
        ],
        out_shape=[
            jax.ShapeDtypeStruct((B, P + L), jnp.int32),
            jax.ShapeDtypeStruct((B, P, D), jnp.float32),
        ],
        scratch_shapes=[
            pltpu.VMEM((P, D), jnp.float32),
            pltpu.VMEM((B, P + L), jnp.int32),
            pltpu.SemaphoreType.DMA,
            pltpu.SemaphoreType.DMA,
            pltpu.SemaphoreType.DMA,
            pltpu.SemaphoreType.DMA,
            pltpu.SemaphoreType.DMA,
        ],
    )(pids2, input_ids, embed_weight)
    return ids_out, emb_out


# speedup vs baseline: 1.3555x; 1.0053x over previous
import functools

import jax
import jax.numpy as jnp
from jax.experimental import pallas as pl
from jax.experimental.pallas import tpu as pltpu

B = 16
L = 2048
P = 2048
D = 1024
PBLK = 128
NBLK = P // PBLK


def _kernel_body(pids_ref, inp_ref, w_ref, ids_out_ref, emb_out_ref,
                 g_ref, ids_scr_ref, sem0, sem1, sem2, sem3, ids_sem):
    sems = (sem0, sem1, sem2, sem3)
    ids_scr_ref[:, 0:P] = jnp.broadcast_to(pids_ref[...], (B, P))
    ids_scr_ref[:, P:P + L] = inp_ref[...]
    ids_copy = pltpu.make_async_copy(ids_scr_ref, ids_out_ref, ids_sem)
    ids_copy.start()

    pending = []
    for j in range(NBLK):
        base = j * PBLK
        idsblk = pids_ref[0:1, base:base + PBLK]
        rows = jax.lax.broadcasted_iota(jnp.int32, (P, PBLK), 0)
        onehot_t = (rows == idsblk).astype(jnp.float32)
        g = jax.lax.dot_general(
            onehot_t, w_ref[...], (((0,), (0,)), ((), ())),
            preferred_element_type=jnp.float32,
        )
        hit = jax.lax.dot_general(
            onehot_t, jnp.ones((P, 1), jnp.float32), (((0,), (0,)), ((), ())),
            preferred_element_type=jnp.float32,
        )
        g_ref[base:base + PBLK, :] = jnp.where(hit > 0.5, g, jnp.float32(jnp.nan))
        fired = []
        for b in range(B):
            c = pltpu.make_async_copy(
                g_ref.at[pl.ds(base, PBLK), :],
                emb_out_ref.at[b, pl.ds(base, PBLK), :],
                sems[b % 4],
            )
            c.start()
            fired.append(c)
        pending.append(fired)
        if len(pending) > 4:
            for c in pending.pop(0):
                c.wait()
    for fired in pending:
        for c in fired:
            c.wait()
    ids_copy.wait()


@functools.partial(jax.jit)
def kernel(input_ids, prompt_ids, embed_weight):
    pids2 = prompt_ids.reshape(1, P)
    ids_out, emb_out = pl.pallas_call(
        _kernel_body,
        in_specs=[
            pl.BlockSpec(memory_space=pltpu.MemorySpace.VMEM),
            pl.BlockSpec(memory_space=pltpu.MemorySpace.VMEM),
            pl.BlockSpec(memory_space=pltpu.MemorySpace.VMEM),
        ],
        out_specs=[
            pl.BlockSpec(memory_space=pl.ANY),
            pl.BlockSpec(memory_space=pl.ANY),
        ],
        out_shape=[
            jax.ShapeDtypeStruct((B, P + L), jnp.int32),
            jax.ShapeDtypeStruct((B, P, D), jnp.float32),
        ],
        scratch_shapes=[
            pltpu.VMEM((P, D), jnp.float32),
            pltpu.VMEM((B, P + L), jnp.int32),
            pltpu.SemaphoreType.DMA,
            pltpu.SemaphoreType.DMA,
            pltpu.SemaphoreType.DMA,
            pltpu.SemaphoreType.DMA,
            pltpu.SemaphoreType.DMA,
        ],
    )(pids2, input_ids, embed_weight)
    return ids_out, emb_out
